# baseline (speedup 1.0000x reference)
import functools

import jax
import jax.numpy as jnp
from jax import lax
from jax.experimental import pallas as pl
from jax.experimental.pallas import tpu as pltpu
from jax.experimental.pallas import tpu_sc as plsc

E = 16384
HIDDEN = 32
PACK = 128 // HIDDEN


def _build_gather():
    info = plsc.get_sparse_core_info()
    nc, ns = info.num_cores, info.num_subcores
    nw = nc * ns
    b_per_w = E // nw
    mesh = plsc.VectorSubcoreMesh(core_axis_name="c", subcore_axis_name="s")

    @functools.partial(
        pl.kernel,
        mesh=mesh,
        out_type=jax.ShapeDtypeStruct((2 * E, 128), jnp.float32),
        scratch_types=[
            pltpu.VMEM((b_per_w,), jnp.int32),
            pltpu.VMEM((b_per_w,), jnp.int32),
            pltpu.VMEM((b_per_w, 128), jnp.float32),
            pltpu.SemaphoreType.DMA,
        ],
    )
    def gather_kernel(row4_hbm, rel4_hbm, obj_w_hbm, rel_w_hbm, out_hbm,
                      row_idx_v, rel_idx_v, rows_v, sem):
        wid = lax.axis_index("s") * nc + lax.axis_index("c")
        base = pl.multiple_of(wid * b_per_w, 128)
        pltpu.sync_copy(row4_hbm.at[pl.ds(base, b_per_w)], row_idx_v)
        pltpu.sync_copy(rel4_hbm.at[pl.ds(base, b_per_w)], rel_idx_v)
        pltpu.async_copy(obj_w_hbm.at[row_idx_v], rows_v, sem).wait()
        pltpu.sync_copy(rows_v, out_hbm.at[pl.ds(base, b_per_w), :])
        pltpu.async_copy(rel_w_hbm.at[rel_idx_v], rows_v, sem).wait()
        pltpu.sync_copy(rows_v, out_hbm.at[pl.ds(E + base, b_per_w), :])

    return gather_kernel


_gather = _build_gather()


def kernel(edge_index, rel, history, obj, rel_table):
    row = edge_index[0]
    obj_w = jnp.concatenate(
        [obj[0::4], obj[1::4], obj[2::4], obj[3::4]], axis=1)
    rel_w = rel_table.reshape(-1, 128)
    wide = _gather(row >> 2, rel >> 2, obj_w, rel_w)
    cls = jnp.concatenate([row & 3, rel & 3])[:, None]
    w4 = wide.reshape(2 * E, PACK, HIDDEN)
    out = jnp.where(
        cls == 0, w4[:, 0, :],
        jnp.where(cls == 1, w4[:, 1, :],
                  jnp.where(cls == 2, w4[:, 2, :], w4[:, 3, :])))
    return out


# speedup vs baseline: 8.2612x; 8.2612x over previous
import functools

import jax
import jax.numpy as jnp
from jax import lax
from jax.experimental import pallas as pl
from jax.experimental.pallas import tpu as pltpu
from jax.experimental.pallas import tpu_sc as plsc

E = 16384
HIDDEN = 32
PACK = 128 // HIDDEN


def _build_gather():
    info = plsc.get_sparse_core_info()
    nc, ns = info.num_cores, info.num_subcores
    nw = nc * ns
    b_per_w = E // nw
    mesh = plsc.VectorSubcoreMesh(core_axis_name="c", subcore_axis_name="s")

    @functools.partial(
        pl.kernel,
        mesh=mesh,
        out_type=jax.ShapeDtypeStruct((2 * E, 128), jnp.float32),
        scratch_types=[
            pltpu.VMEM((b_per_w,), jnp.int32),
            pltpu.VMEM((b_per_w,), jnp.int32),
            pltpu.VMEM((b_per_w, 128), jnp.float32),
            pltpu.SemaphoreType.DMA,
        ],
    )
    def gather_kernel(row4_hbm, rel4_hbm, obj_w_hbm, rel_w_hbm, out_hbm,
                      row_idx_v, rel_idx_v, rows_v, sem):
        wid = lax.axis_index("s") * nc + lax.axis_index("c")
        base = pl.multiple_of(wid * b_per_w, 128)
        pltpu.sync_copy(row4_hbm.at[pl.ds(base, b_per_w)], row_idx_v)
        pltpu.sync_copy(rel4_hbm.at[pl.ds(base, b_per_w)], rel_idx_v)
        pltpu.async_copy(obj_w_hbm.at[row_idx_v], rows_v, sem).wait()
        pltpu.sync_copy(rows_v, out_hbm.at[pl.ds(base, b_per_w), :])
        pltpu.async_copy(rel_w_hbm.at[rel_idx_v], rows_v, sem).wait()
        pltpu.sync_copy(rows_v, out_hbm.at[pl.ds(E + base, b_per_w), :])

    return gather_kernel


_gather = _build_gather()


def kernel(edge_index, rel, history, obj, rel_table):
    row = edge_index[0]
    obj_w = obj.reshape(-1, 128)
    rel_w = rel_table.reshape(-1, 128)
    wide = _gather(row >> 2, rel >> 2, obj_w, rel_w)
    cls = jnp.concatenate([row & 3, rel & 3])[:, None]
    w4 = wide.reshape(2 * E, PACK, HIDDEN)
    out = jnp.where(
        cls == 0, w4[:, 0, :],
        jnp.where(cls == 1, w4[:, 1, :],
                  jnp.where(cls == 2, w4[:, 2, :], w4[:, 3, :])))
    return out


# speedup vs baseline: 8.5265x; 1.0321x over previous
import functools

import jax
import jax.numpy as jnp
from jax import lax
from jax.experimental import pallas as pl
from jax.experimental.pallas import tpu as pltpu
from jax.experimental.pallas import tpu_sc as plsc

E = 16384
HIDDEN = 32


def _build_gather():
    info = plsc.get_sparse_core_info()
    nc, ns = info.num_cores, info.num_subcores
    nw = nc * ns
    b_per_w = E // nw
    mesh = plsc.VectorSubcoreMesh(core_axis_name="c", subcore_axis_name="s")

    @functools.partial(
        pl.kernel,
        mesh=mesh,
        out_type=jax.ShapeDtypeStruct((2 * E, HIDDEN), jnp.float32),
        compiler_params=pltpu.CompilerParams(use_tc_tiling_on_sc=False),
        scratch_types=[
            pltpu.VMEM((b_per_w,), jnp.int32),
            pltpu.VMEM((b_per_w,), jnp.int32),
            pltpu.VMEM((b_per_w, HIDDEN), jnp.float32),
            pltpu.VMEM((b_per_w, HIDDEN), jnp.float32),
            pltpu.SemaphoreType.DMA,
            pltpu.SemaphoreType.DMA,
        ],
    )
    def gather_kernel(row_hbm, rel_hbm, obj_hbm, rel_table_hbm, out_hbm,
                      row_idx_v, rel_idx_v, obj_rows_v, rel_rows_v,
                      sem_obj, sem_rel):
        wid = lax.axis_index("s") * nc + lax.axis_index("c")
        base = wid * b_per_w
        pltpu.sync_copy(row_hbm.at[pl.ds(base, b_per_w)], row_idx_v)
        pltpu.sync_copy(rel_hbm.at[pl.ds(base, b_per_w)], rel_idx_v)
        obj_cp = pltpu.async_copy(obj_hbm.at[row_idx_v], obj_rows_v, sem_obj)
        rel_cp = pltpu.async_copy(rel_table_hbm.at[rel_idx_v], rel_rows_v,
                                  sem_rel)
        obj_cp.wait()
        pltpu.sync_copy(obj_rows_v, out_hbm.at[pl.ds(base, b_per_w)])
        rel_cp.wait()
        pltpu.sync_copy(rel_rows_v, out_hbm.at[pl.ds(E + base, b_per_w)])

    return gather_kernel


_gather = _build_gather()


def kernel(edge_index, rel, history, obj, rel_table):
    row = edge_index[0]
    return _gather(row, rel, obj, rel_table)
